# Initial kernel scaffold; baseline (speedup 1.0000x reference)
import jax
import jax.numpy as jnp
from jax.experimental import pallas as pl


def kernel(x, edge_index, edge_attr, W_front, b_front, W1, b1):
    raise NotImplementedError("write your pallas kernel here")


# speedup vs baseline: 16.4344x; 16.4344x over previous
import functools

import jax
import jax.numpy as jnp
from jax import lax
from jax.experimental import pallas as pl
from jax.experimental.pallas import tpu as pltpu
from jax.experimental.pallas import tpu_sc as plsc

N = 50000
E = 800000
FEAT = 128
C0 = 64

NC = 2
NS = 16
L = 16

NPAD = 50176
R = NPAD // NS
PER_TILE = E // NS
CE = 2000
NCH = PER_TILE // CE


def _dense_body(x0, x1, x2, x3, wf, bf, w1, b1r, o_ref):
    s = x0[...] + x1[...] + x2[...] + x3[...]
    h = jnp.dot(s, wf[...], preferred_element_type=jnp.float32,
                precision=lax.Precision.HIGHEST) + bf[...]
    w1m = w1[...]
    wa = w1m[:C0] - w1m[C0:]
    wb = w1m[C0:]
    wcat = jnp.concatenate([wa, wb], axis=1)
    bcat = jnp.concatenate([b1r[...], jnp.zeros((1, 2), jnp.float32)], axis=1)
    o_ref[...] = jnp.dot(h, wcat, preferred_element_type=jnp.float32,
                         precision=lax.Precision.HIGHEST) + bcat


def _dense_tables(x, W_front, b_front, W1, b1):
    BN = 2048
    grid = (pl.cdiv(N, BN),)
    full = lambda i: (0, 0)
    specs = [
        pl.BlockSpec((BN, FEAT), lambda i, c=c: (i, c)) for c in (0, 1, 2, 4)
    ]
    specs += [
        pl.BlockSpec((FEAT, C0), full),
        pl.BlockSpec((1, C0), full),
        pl.BlockSpec((FEAT, 2), full),
        pl.BlockSpec((1, 2), full),
    ]
    return pl.pallas_call(
        _dense_body,
        grid=grid,
        in_specs=specs,
        out_specs=pl.BlockSpec((BN, 4), lambda i: (i, 0)),
        out_shape=jax.ShapeDtypeStruct((N, 4), jnp.float32),
    )(x, x, x, x, W_front, b_front.reshape(1, C0), W1, b1.reshape(1, 2))


def _sc_body(pt_hbm, src_hbm, dst_hbm, attr_hbm, out_hbm, stage_hbm,
             m_v, bk_v, sb0, sb1, db0, db1, ab0, ab1,
             racc, rtmp, sem0, sem1):
    c = lax.axis_index("c")
    s = lax.axis_index("s")

    @pl.when(c == 0)
    def _():
        pltpu.sync_copy(pt_hbm.at[pl.ds(2 * NPAD, NPAD)], bk_v)

    @pl.when(c == 1)
    def _():
        pltpu.sync_copy(pt_hbm.at[pl.ds(3 * NPAD, NPAD)], bk_v)

    def init_body(i, _):
        m_v[pl.ds(i * L, L)] = jnp.full((L,), -jnp.inf, jnp.float32)
        return 0
    lax.fori_loop(0, NPAD // L, init_body, 0)

    base = s * PER_TILE
    sbufs = (sb0, sb1)
    dbufs = (db0, db1)
    abufs = (ab0, ab1)
    sems = (sem0, sem1)

    def start_chunk(i, slot):
        off = base + i * CE
        return (
            pltpu.async_copy(src_hbm.at[pl.ds(off, CE)], sbufs[slot], sems[slot]),
            pltpu.async_copy(dst_hbm.at[pl.ds(off, CE)], dbufs[slot], sems[slot]),
            pltpu.async_copy(attr_hbm.at[pl.ds(off, CE)], abufs[slot], sems[slot]),
        )

    def process_chunk(slot):
        sbuf, dbuf, abuf = sbufs[slot], dbufs[slot], abufs[slot]

        def group_body(g, _):
            off = g * L
            sids = sbuf[pl.ds(off, L)]
            dids = dbuf[pl.ds(off, L)]
            at = abuf[pl.ds(off, L)]
            keep = (at == 111.0) | (at == 0.0)
            val = plsc.load_gather(bk_v, [sids])

            cur = plsc.load_gather(m_v, [dids])
            need0 = keep & (val > cur)

            def cond(need):
                return jnp.any(need)

            def body(need):
                plsc.store_scatter(m_v, [dids], val, mask=need)
                cur = plsc.load_gather(m_v, [dids])
                return need & (val > cur)

            lax.while_loop(cond, body, need0)
            return 0

        lax.fori_loop(0, CE // L, group_body, 0)

    pend = {0: start_chunk(0, 0), 1: None}
    for i in range(NCH):
        slot = i % 2
        for d in pend[slot]:
            d.wait()
        if i + 1 < NCH:
            pend[(i + 1) % 2] = start_chunk(i + 1, (i + 1) % 2)
        process_chunk(slot)

    sbase = (c * NS + s) * NPAD
    pltpu.sync_copy(m_v, stage_hbm.at[pl.ds(sbase, NPAD)])
    plsc.subcore_barrier()

    rbase = s * R
    cbase = c * NS * NPAD
    pltpu.sync_copy(stage_hbm.at[pl.ds(cbase + rbase, R)], racc)
    for t in range(1, NS):
        pltpu.sync_copy(stage_hbm.at[pl.ds(cbase + t * NPAD + rbase, R)], rtmp)

        def red_body(j, _):
            sl = pl.ds(j * L, L)
            racc[sl] = jnp.maximum(racc[sl], rtmp[sl])
            return 0
        lax.fori_loop(0, R // L, red_body, 0)

    @pl.when(c == 0)
    def _():
        pltpu.sync_copy(pt_hbm.at[pl.ds(rbase, R)], rtmp)

    @pl.when(c == 1)
    def _():
        pltpu.sync_copy(pt_hbm.at[pl.ds(NPAD + rbase, R)], rtmp)

    def comb_body(j, _):
        sl = pl.ds(j * L, L)
        mv = racc[sl]
        racc[sl] = jnp.where(mv == -jnp.inf, 0.0, rtmp[sl] + mv)
        return 0
    lax.fori_loop(0, R // L, comb_body, 0)

    @pl.when(c == 0)
    def _():
        pltpu.sync_copy(racc, out_hbm.at[pl.ds(rbase, R)])

    @pl.when(c == 1)
    def _():
        pltpu.sync_copy(racc, out_hbm.at[pl.ds(NPAD + rbase, R)])


_sc_call = pl.kernel(
    _sc_body,
    out_type=(
        jax.ShapeDtypeStruct((2 * NPAD,), jnp.float32),
        jax.ShapeDtypeStruct((NC * NS * NPAD,), jnp.float32),
    ),
    mesh=plsc.VectorSubcoreMesh(
        core_axis_name="c", subcore_axis_name="s", num_cores=NC, num_subcores=NS
    ),
    compiler_params=pltpu.CompilerParams(needs_layout_passes=False),
    scratch_types=[
        pltpu.VMEM((NPAD,), jnp.float32),
        pltpu.VMEM((NPAD,), jnp.float32),
        pltpu.VMEM((CE,), jnp.int32),
        pltpu.VMEM((CE,), jnp.int32),
        pltpu.VMEM((CE,), jnp.int32),
        pltpu.VMEM((CE,), jnp.int32),
        pltpu.VMEM((CE,), jnp.float32),
        pltpu.VMEM((CE,), jnp.float32),
        pltpu.VMEM((R,), jnp.float32),
        pltpu.VMEM((R,), jnp.float32),
        pltpu.SemaphoreType.DMA,
        pltpu.SemaphoreType.DMA,
    ],
)


def kernel(x, edge_index, edge_attr, W_front, b_front, W1, b1):
    p = _dense_tables(x, W_front, b_front, W1, b1)
    pt = jnp.pad(p.T, ((0, 0), (0, NPAD - N))).reshape(-1)
    out_t, _ = _sc_call(pt, edge_index[0], edge_index[1], edge_attr)
    return out_t.reshape(2, NPAD)[:, :N].T


# speedup vs baseline: 17.1493x; 1.0435x over previous
import functools

import jax
import jax.numpy as jnp
from jax import lax
from jax.experimental import pallas as pl
from jax.experimental.pallas import tpu as pltpu
from jax.experimental.pallas import tpu_sc as plsc

N = 50000
E = 800000
FEAT = 128
C0 = 64

NC = 2
NS = 16
L = 16

NPAD = 50176
R = NPAD // NS
PER_TILE = E // NS
CE = 2000
NCH = PER_TILE // CE


def _dense_body(x0, x1, x2, x3, wf, bf, w1, b1c, o_ref):
    s = x0[...] + x1[...] + x2[...] + x3[...]
    h = jnp.dot(s, wf[...], preferred_element_type=jnp.float32,
                precision=lax.Precision.HIGHEST) + bf[...]
    w1m = w1[...]
    wa = w1m[:C0] - w1m[C0:]
    wb = w1m[C0:]
    wcat = jnp.concatenate([wa, wb], axis=1)
    bcat = jnp.concatenate([b1c[...], jnp.zeros((2, 1), jnp.float32)], axis=0)
    o_ref[...] = lax.dot_general(
        wcat, h, (((0,), (1,)), ((), ())),
        preferred_element_type=jnp.float32,
        precision=lax.Precision.HIGHEST) + bcat


def _dense_tables(x, W_front, b_front, W1, b1):
    BN = 3584
    grid = (NPAD // BN,)
    full = lambda i: (0, 0)
    specs = [
        pl.BlockSpec((BN, FEAT), lambda i, c=c: (i, c)) for c in (0, 1, 2, 4)
    ]
    specs += [
        pl.BlockSpec((FEAT, C0), full),
        pl.BlockSpec((1, C0), full),
        pl.BlockSpec((FEAT, 2), full),
        pl.BlockSpec((2, 1), full),
    ]
    return pl.pallas_call(
        _dense_body,
        grid=grid,
        in_specs=specs,
        out_specs=pl.BlockSpec((4, BN), lambda i: (0, i)),
        out_shape=jax.ShapeDtypeStruct((4, NPAD), jnp.float32),
    )(x, x, x, x, W_front, b_front.reshape(1, C0), W1, b1.reshape(2, 1))


def _sc_body(pt_hbm, src_hbm, dst_hbm, attr_hbm, out_hbm, stage_hbm,
             m_v, bk_v, sb0, sb1, db0, db1, ab0, ab1,
             racc, rtmp, sem0, sem1):
    c = lax.axis_index("c")
    s = lax.axis_index("s")

    @pl.when(c == 0)
    def _():
        pltpu.sync_copy(pt_hbm.at[pl.ds(2 * NPAD, NPAD)], bk_v)

    @pl.when(c == 1)
    def _():
        pltpu.sync_copy(pt_hbm.at[pl.ds(3 * NPAD, NPAD)], bk_v)

    def init_body(i, _):
        m_v[pl.ds(i * L, L)] = jnp.full((L,), -jnp.inf, jnp.float32)
        return 0
    lax.fori_loop(0, NPAD // L, init_body, 0)

    base = s * PER_TILE
    sbufs = (sb0, sb1)
    dbufs = (db0, db1)
    abufs = (ab0, ab1)
    sems = (sem0, sem1)

    def start_chunk(i, slot):
        off = base + i * CE
        return (
            pltpu.async_copy(src_hbm.at[pl.ds(off, CE)], sbufs[slot], sems[slot]),
            pltpu.async_copy(dst_hbm.at[pl.ds(off, CE)], dbufs[slot], sems[slot]),
            pltpu.async_copy(attr_hbm.at[pl.ds(off, CE)], abufs[slot], sems[slot]),
        )

    def process_chunk(slot):
        sbuf, dbuf, abuf = sbufs[slot], dbufs[slot], abufs[slot]

        def group_body(g, _):
            off = g * L
            sids = sbuf[pl.ds(off, L)]
            dids = dbuf[pl.ds(off, L)]
            at = abuf[pl.ds(off, L)]
            keep = (at == 111.0) | (at == 0.0)
            val = plsc.load_gather(bk_v, [sids])

            cur = plsc.load_gather(m_v, [dids])
            need0 = keep & (val > cur)

            def cond(need):
                return jnp.any(need)

            def body(need):
                plsc.store_scatter(m_v, [dids], val, mask=need)
                cur = plsc.load_gather(m_v, [dids])
                return need & (val > cur)

            lax.while_loop(cond, body, need0)
            return 0

        lax.fori_loop(0, CE // L, group_body, 0)

    pend = {0: start_chunk(0, 0), 1: None}
    for i in range(NCH):
        slot = i % 2
        for d in pend[slot]:
            d.wait()
        if i + 1 < NCH:
            pend[(i + 1) % 2] = start_chunk(i + 1, (i + 1) % 2)
        process_chunk(slot)

    sbase = (c * NS + s) * NPAD
    pltpu.sync_copy(m_v, stage_hbm.at[pl.ds(sbase, NPAD)])
    plsc.subcore_barrier()

    rbase = s * R
    cbase = c * NS * NPAD
    pltpu.sync_copy(stage_hbm.at[pl.ds(cbase + rbase, R)], racc)
    for t in range(1, NS):
        pltpu.sync_copy(stage_hbm.at[pl.ds(cbase + t * NPAD + rbase, R)], rtmp)

        def red_body(j, _):
            sl = pl.ds(j * L, L)
            racc[sl] = jnp.maximum(racc[sl], rtmp[sl])
            return 0
        lax.fori_loop(0, R // L, red_body, 0)

    @pl.when(c == 0)
    def _():
        pltpu.sync_copy(pt_hbm.at[pl.ds(rbase, R)], rtmp)

    @pl.when(c == 1)
    def _():
        pltpu.sync_copy(pt_hbm.at[pl.ds(NPAD + rbase, R)], rtmp)

    def comb_body(j, _):
        sl = pl.ds(j * L, L)
        mv = racc[sl]
        racc[sl] = jnp.where(mv == -jnp.inf, 0.0, rtmp[sl] + mv)
        return 0
    lax.fori_loop(0, R // L, comb_body, 0)

    @pl.when(c == 0)
    def _():
        pltpu.sync_copy(racc, out_hbm.at[pl.ds(rbase, R)])

    @pl.when(c == 1)
    def _():
        pltpu.sync_copy(racc, out_hbm.at[pl.ds(NPAD + rbase, R)])


_sc_call = pl.kernel(
    _sc_body,
    out_type=(
        jax.ShapeDtypeStruct((2 * NPAD,), jnp.float32),
        jax.ShapeDtypeStruct((NC * NS * NPAD,), jnp.float32),
    ),
    mesh=plsc.VectorSubcoreMesh(
        core_axis_name="c", subcore_axis_name="s", num_cores=NC, num_subcores=NS
    ),
    compiler_params=pltpu.CompilerParams(needs_layout_passes=False),
    scratch_types=[
        pltpu.VMEM((NPAD,), jnp.float32),
        pltpu.VMEM((NPAD,), jnp.float32),
        pltpu.VMEM((CE,), jnp.int32),
        pltpu.VMEM((CE,), jnp.int32),
        pltpu.VMEM((CE,), jnp.int32),
        pltpu.VMEM((CE,), jnp.int32),
        pltpu.VMEM((CE,), jnp.float32),
        pltpu.VMEM((CE,), jnp.float32),
        pltpu.VMEM((R,), jnp.float32),
        pltpu.VMEM((R,), jnp.float32),
        pltpu.SemaphoreType.DMA,
        pltpu.SemaphoreType.DMA,
    ],
)


def kernel(x, edge_index, edge_attr, W_front, b_front, W1, b1):
    pt = _dense_tables(x, W_front, b_front, W1, b1).reshape(-1)
    out_t, _ = _sc_call(pt, edge_index[0], edge_index[1], edge_attr)
    return out_t.reshape(2, NPAD)[:, :N].T


# speedup vs baseline: 21.2746x; 1.2406x over previous
import functools

import jax
import jax.numpy as jnp
from jax import lax
from jax.experimental import pallas as pl
from jax.experimental.pallas import tpu as pltpu
from jax.experimental.pallas import tpu_sc as plsc

N = 50000
E = 800000
FEAT = 128
C0 = 64

NC = 2
NS = 16
L = 16

NPAD = 50176
R = NPAD // NS
PER_TILE = E // NS
CE = 2000
NCH = PER_TILE // CE


def _dense_body(x0, x1, x2, x3, wf, bf, w1, b1c, o_ref):
    s = x0[...] + x1[...] + x2[...] + x3[...]
    h = jnp.dot(s, wf[...], preferred_element_type=jnp.float32,
                precision=lax.Precision.HIGHEST) + bf[...]
    w1m = w1[...]
    wa = w1m[:C0] - w1m[C0:]
    wb = w1m[C0:]
    wcat = jnp.concatenate([wa, wb], axis=1)
    bcat = jnp.concatenate([b1c[...], jnp.zeros((2, 1), jnp.float32)], axis=0)
    o_ref[...] = lax.dot_general(
        wcat, h, (((0,), (1,)), ((), ())),
        preferred_element_type=jnp.float32,
        precision=lax.Precision.HIGHEST) + bcat


def _dense_tables(x, W_front, b_front, W1, b1):
    BN = 3584
    grid = (NPAD // BN,)
    full = lambda i: (0, 0)
    specs = [
        pl.BlockSpec((BN, FEAT), lambda i, c=c: (i, c)) for c in (0, 1, 2, 4)
    ]
    specs += [
        pl.BlockSpec((FEAT, C0), full),
        pl.BlockSpec((1, C0), full),
        pl.BlockSpec((FEAT, 2), full),
        pl.BlockSpec((2, 1), full),
    ]
    return pl.pallas_call(
        _dense_body,
        grid=grid,
        in_specs=specs,
        out_specs=pl.BlockSpec((4, BN), lambda i: (0, i)),
        out_shape=jax.ShapeDtypeStruct((4, NPAD), jnp.float32),
    )(x, x, x, x, W_front, b_front.reshape(1, C0), W1, b1.reshape(2, 1))


def _sc_body(pt_hbm, src_hbm, dst_hbm, attr_hbm, out_hbm, stage_hbm,
             m_v, bk_v, sb0, sb1, db0, db1, ab0, ab1,
             racc, rtmp, sem0, sem1):
    c = lax.axis_index("c")
    s = lax.axis_index("s")

    @pl.when(c == 0)
    def _():
        pltpu.sync_copy(pt_hbm.at[pl.ds(2 * NPAD, NPAD)], bk_v)

    @pl.when(c == 1)
    def _():
        pltpu.sync_copy(pt_hbm.at[pl.ds(3 * NPAD, NPAD)], bk_v)

    def init_body(i, _):
        m_v[pl.ds(i * L, L)] = jnp.full((L,), -jnp.inf, jnp.float32)
        return 0
    lax.fori_loop(0, NPAD // L, init_body, 0)

    base = s * PER_TILE
    sbufs = (sb0, sb1)
    dbufs = (db0, db1)
    abufs = (ab0, ab1)
    sems = (sem0, sem1)

    def start_chunk(i, slot):
        off = base + i * CE
        return (
            pltpu.async_copy(src_hbm.at[pl.ds(off, CE)], sbufs[slot], sems[slot]),
            pltpu.async_copy(dst_hbm.at[pl.ds(off, CE)], dbufs[slot], sems[slot]),
            pltpu.async_copy(attr_hbm.at[pl.ds(off, CE)], abufs[slot], sems[slot]),
        )

    def process_chunk(slot):
        sbuf, dbuf, abuf = sbufs[slot], dbufs[slot], abufs[slot]

        iota = lax.iota(jnp.int32, L)

        def group_body(g, _):
            off = g * L
            sids = sbuf[pl.ds(off, L)]
            dids = dbuf[pl.ds(off, L)]
            at = abuf[pl.ds(off, L)]
            keep = (at == 111.0) | (at == 0.0)
            val = plsc.load_gather(bk_v, [sids])
            valm = jnp.where(keep, val, -jnp.inf)

            dk, dv = plsc.sort_key_val(dids, valm)
            for k in (1, 2, 4, 8):
                idxk = jnp.maximum(iota - k, 0)
                pk = dk.at[idxk].get(mode="promise_in_bounds")
                pv = dv.at[idxk].get(mode="promise_in_bounds")
                same = (pk == dk) & (iota >= k)
                dv = jnp.where(same, jnp.maximum(dv, pv), dv)
            nxt = dk.at[jnp.minimum(iota + 1, L - 1)].get(mode="promise_in_bounds")
            is_last = (dk != nxt) | (iota == L - 1)
            cur = plsc.load_gather(m_v, [dk])
            need = is_last & (dv > cur)
            plsc.store_scatter(m_v, [dk], dv, mask=need)
            return 0

        lax.fori_loop(0, CE // L, group_body, 0)

    pend = {0: start_chunk(0, 0), 1: None}
    for i in range(NCH):
        slot = i % 2
        for d in pend[slot]:
            d.wait()
        if i + 1 < NCH:
            pend[(i + 1) % 2] = start_chunk(i + 1, (i + 1) % 2)
        process_chunk(slot)

    sbase = (c * NS + s) * NPAD
    pltpu.sync_copy(m_v, stage_hbm.at[pl.ds(sbase, NPAD)])
    plsc.subcore_barrier()

    rbase = s * R
    cbase = c * NS * NPAD
    pltpu.sync_copy(stage_hbm.at[pl.ds(cbase + rbase, R)], racc)
    for t in range(1, NS):
        pltpu.sync_copy(stage_hbm.at[pl.ds(cbase + t * NPAD + rbase, R)], rtmp)

        def red_body(j, _):
            sl = pl.ds(j * L, L)
            racc[sl] = jnp.maximum(racc[sl], rtmp[sl])
            return 0
        lax.fori_loop(0, R // L, red_body, 0)

    @pl.when(c == 0)
    def _():
        pltpu.sync_copy(pt_hbm.at[pl.ds(rbase, R)], rtmp)

    @pl.when(c == 1)
    def _():
        pltpu.sync_copy(pt_hbm.at[pl.ds(NPAD + rbase, R)], rtmp)

    def comb_body(j, _):
        sl = pl.ds(j * L, L)
        mv = racc[sl]
        racc[sl] = jnp.where(mv == -jnp.inf, 0.0, rtmp[sl] + mv)
        return 0
    lax.fori_loop(0, R // L, comb_body, 0)

    @pl.when(c == 0)
    def _():
        pltpu.sync_copy(racc, out_hbm.at[pl.ds(rbase, R)])

    @pl.when(c == 1)
    def _():
        pltpu.sync_copy(racc, out_hbm.at[pl.ds(NPAD + rbase, R)])


_sc_call = pl.kernel(
    _sc_body,
    out_type=(
        jax.ShapeDtypeStruct((2 * NPAD,), jnp.float32),
        jax.ShapeDtypeStruct((NC * NS * NPAD,), jnp.float32),
    ),
    mesh=plsc.VectorSubcoreMesh(
        core_axis_name="c", subcore_axis_name="s", num_cores=NC, num_subcores=NS
    ),
    compiler_params=pltpu.CompilerParams(needs_layout_passes=False),
    scratch_types=[
        pltpu.VMEM((NPAD,), jnp.float32),
        pltpu.VMEM((NPAD,), jnp.float32),
        pltpu.VMEM((CE,), jnp.int32),
        pltpu.VMEM((CE,), jnp.int32),
        pltpu.VMEM((CE,), jnp.int32),
        pltpu.VMEM((CE,), jnp.int32),
        pltpu.VMEM((CE,), jnp.float32),
        pltpu.VMEM((CE,), jnp.float32),
        pltpu.VMEM((R,), jnp.float32),
        pltpu.VMEM((R,), jnp.float32),
        pltpu.SemaphoreType.DMA,
        pltpu.SemaphoreType.DMA,
    ],
)


def kernel(x, edge_index, edge_attr, W_front, b_front, W1, b1):
    pt = _dense_tables(x, W_front, b_front, W1, b1).reshape(-1)
    out_t, _ = _sc_call(pt, edge_index[0], edge_index[1], edge_attr)
    return out_t.reshape(2, NPAD)[:, :N].T


# speedup vs baseline: 21.3266x; 1.0024x over previous
import functools

import jax
import jax.numpy as jnp
from jax import lax
from jax.experimental import pallas as pl
from jax.experimental.pallas import tpu as pltpu
from jax.experimental.pallas import tpu_sc as plsc

N = 50000
E = 800000
FEAT = 128
C0 = 64

NC = 2
NS = 16
L = 16

NPAD = 50176
R = NPAD // NS
PER_TILE = E // NS
CE = 2000
NCH = PER_TILE // CE
NRED = 4
Q = NPAD // NRED
Q16 = Q // NS


def _dense_body(x0, x1, x2, x3, wf, bf, w1, b1c, o_ref):
    s = x0[...] + x1[...] + x2[...] + x3[...]
    h = jnp.dot(s, wf[...], preferred_element_type=jnp.float32,
                precision=lax.Precision.HIGHEST) + bf[...]
    w1m = w1[...]
    wa = w1m[:C0] - w1m[C0:]
    wb = w1m[C0:]
    wcat = jnp.concatenate([wa, wb], axis=1)
    bcat = jnp.concatenate([b1c[...], jnp.zeros((2, 1), jnp.float32)], axis=0)
    o_ref[...] = lax.dot_general(
        wcat, h, (((0,), (1,)), ((), ())),
        preferred_element_type=jnp.float32,
        precision=lax.Precision.HIGHEST) + bcat


def _dense_tables(x, W_front, b_front, W1, b1):
    BN = 3584
    grid = (NPAD // BN,)
    full = lambda i: (0, 0)
    specs = [
        pl.BlockSpec((BN, FEAT), lambda i, c=c: (i, c)) for c in (0, 1, 2, 4)
    ]
    specs += [
        pl.BlockSpec((FEAT, C0), full),
        pl.BlockSpec((1, C0), full),
        pl.BlockSpec((FEAT, 2), full),
        pl.BlockSpec((2, 1), full),
    ]
    return pl.pallas_call(
        _dense_body,
        grid=grid,
        in_specs=specs,
        out_specs=pl.BlockSpec((4, BN), lambda i: (0, i)),
        out_shape=jax.ShapeDtypeStruct((4, NPAD), jnp.float32),
    )(x, x, x, x, W_front, b_front.reshape(1, C0), W1, b1.reshape(2, 1))


def _sc_body(pt_hbm, src_hbm, dst_hbm, attr_hbm, out_hbm,
             m_v, bk_v, sb0, sb1, db0, db1, ab0, ab1,
             racc, rtmp, stage_sp, sem0, sem1):
    c = lax.axis_index("c")
    s = lax.axis_index("s")

    @pl.when(c == 0)
    def _():
        pltpu.sync_copy(pt_hbm.at[pl.ds(2 * NPAD, NPAD)], bk_v)

    @pl.when(c == 1)
    def _():
        pltpu.sync_copy(pt_hbm.at[pl.ds(3 * NPAD, NPAD)], bk_v)

    def init_body(i, _):
        m_v[pl.ds(i * L, L)] = jnp.full((L,), -jnp.inf, jnp.float32)
        return 0
    lax.fori_loop(0, NPAD // L, init_body, 0)

    base = s * PER_TILE
    sbufs = (sb0, sb1)
    dbufs = (db0, db1)
    abufs = (ab0, ab1)
    sems = (sem0, sem1)

    def start_chunk(i, slot):
        off = base + i * CE
        return (
            pltpu.async_copy(src_hbm.at[pl.ds(off, CE)], sbufs[slot], sems[slot]),
            pltpu.async_copy(dst_hbm.at[pl.ds(off, CE)], dbufs[slot], sems[slot]),
            pltpu.async_copy(attr_hbm.at[pl.ds(off, CE)], abufs[slot], sems[slot]),
        )

    def process_chunk(slot):
        sbuf, dbuf, abuf = sbufs[slot], dbufs[slot], abufs[slot]

        iota = lax.iota(jnp.int32, L)

        def group_body(g, _):
            off = g * L
            sids = sbuf[pl.ds(off, L)]
            dids = dbuf[pl.ds(off, L)]
            at = abuf[pl.ds(off, L)]
            keep = (at == 111.0) | (at == 0.0)
            val = plsc.load_gather(bk_v, [sids])
            valm = jnp.where(keep, val, -jnp.inf)

            dk, dv = plsc.sort_key_val(dids, valm)
            for k in (1, 2, 4, 8):
                idxk = jnp.maximum(iota - k, 0)
                pk = dk.at[idxk].get(mode="promise_in_bounds")
                pv = dv.at[idxk].get(mode="promise_in_bounds")
                same = (pk == dk) & (iota >= k)
                dv = jnp.where(same, jnp.maximum(dv, pv), dv)
            nxt = dk.at[jnp.minimum(iota + 1, L - 1)].get(mode="promise_in_bounds")
            is_last = (dk != nxt) | (iota == L - 1)
            cur = plsc.load_gather(m_v, [dk])
            need = is_last & (dv > cur)
            plsc.store_scatter(m_v, [dk], dv, mask=need)
            return 0

        lax.fori_loop(0, CE // L, group_body, 0)

    pend = {0: start_chunk(0, 0), 1: None}
    for i in range(NCH):
        slot = i % 2
        for d in pend[slot]:
            d.wait()
        if i + 1 < NCH:
            pend[(i + 1) % 2] = start_chunk(i + 1, (i + 1) % 2)
        process_chunk(slot)

    for r in range(NRED):
        pltpu.sync_copy(m_v.at[pl.ds(r * Q, Q)], stage_sp.at[pl.ds(s * Q, Q)])
        plsc.subcore_barrier()

        ob = r * Q + s * Q16
        pltpu.sync_copy(stage_sp.at[pl.ds(s * Q16, Q16)], racc)
        for t in range(1, NS):
            pltpu.sync_copy(stage_sp.at[pl.ds(t * Q + s * Q16, Q16)], rtmp)

            def red_body(j, _):
                sl = pl.ds(j * L, L)
                racc[sl] = jnp.maximum(racc[sl], rtmp[sl])
                return 0
            lax.fori_loop(0, Q16 // L, red_body, 0)

        @pl.when(c == 0)
        def _():
            pltpu.sync_copy(pt_hbm.at[pl.ds(ob, Q16)], rtmp)

        @pl.when(c == 1)
        def _():
            pltpu.sync_copy(pt_hbm.at[pl.ds(NPAD + ob, Q16)], rtmp)

        def comb_body(j, _):
            sl = pl.ds(j * L, L)
            mv = racc[sl]
            racc[sl] = jnp.where(mv == -jnp.inf, 0.0, rtmp[sl] + mv)
            return 0
        lax.fori_loop(0, Q16 // L, comb_body, 0)

        @pl.when(c == 0)
        def _():
            pltpu.sync_copy(racc, out_hbm.at[pl.ds(ob, Q16)])

        @pl.when(c == 1)
        def _():
            pltpu.sync_copy(racc, out_hbm.at[pl.ds(NPAD + ob, Q16)])

        plsc.subcore_barrier()


_sc_call = pl.kernel(
    _sc_body,
    out_type=jax.ShapeDtypeStruct((2 * NPAD,), jnp.float32),
    mesh=plsc.VectorSubcoreMesh(
        core_axis_name="c", subcore_axis_name="s", num_cores=NC, num_subcores=NS
    ),
    compiler_params=pltpu.CompilerParams(needs_layout_passes=False),
    scratch_types=[
        pltpu.VMEM((NPAD,), jnp.float32),
        pltpu.VMEM((NPAD,), jnp.float32),
        pltpu.VMEM((CE,), jnp.int32),
        pltpu.VMEM((CE,), jnp.int32),
        pltpu.VMEM((CE,), jnp.int32),
        pltpu.VMEM((CE,), jnp.int32),
        pltpu.VMEM((CE,), jnp.float32),
        pltpu.VMEM((CE,), jnp.float32),
        pltpu.VMEM((Q16,), jnp.float32),
        pltpu.VMEM((Q16,), jnp.float32),
        pltpu.VMEM_SHARED((NS * Q,), jnp.float32),
        pltpu.SemaphoreType.DMA,
        pltpu.SemaphoreType.DMA,
    ],
)


def kernel(x, edge_index, edge_attr, W_front, b_front, W1, b1):
    pt = _dense_tables(x, W_front, b_front, W1, b1).reshape(-1)
    out_t = _sc_call(pt, edge_index[0], edge_index[1], edge_attr)
    return out_t.reshape(2, NPAD)[:, :N].T
